# baseline (speedup 1.0000x reference)
import functools

import jax
import jax.numpy as jnp
from jax import lax
from jax.experimental import pallas as pl
from jax.experimental.pallas import tpu as pltpu
from jax.experimental.pallas import tpu_sc as plsc

B = 16384
D = 64
K = 3
NC = 2
NS = 16
NW = NC * NS
PER_W = B // NW
CHUNK = 128
NCHUNK = PER_W // CHUNK
W = 2 * D
TROWS = 500000
NBLK = 3907


def _tp_body(a_ref, b_ref, o_ref):
    ident = (lax.broadcasted_iota(jnp.int32, (D, D), 0) ==
             lax.broadcasted_iota(jnp.int32, (D, D), 1)).astype(jnp.float32)
    dn = (((0,), (0,)), ((), ()))
    ta = lax.dot_general(a_ref[...], ident, dn,
                         preferred_element_type=jnp.float32,
                         precision=lax.Precision.HIGHEST)
    tb = lax.dot_general(b_ref[...], ident, dn,
                         preferred_element_type=jnp.float32,
                         precision=lax.Precision.HIGHEST)
    o_ref[...] = jnp.concatenate([ta, tb], axis=1)


_transpose = pl.pallas_call(
    _tp_body,
    grid=(NBLK,),
    in_specs=[
        pl.BlockSpec((D, 128), lambda j: (0, 2 * j)),
        pl.BlockSpec((D, 128), lambda j: (0, 2 * j + 1)),
    ],
    out_specs=pl.BlockSpec((128, W), lambda j: (j, 0)),
    out_shape=jax.ShapeDtypeStruct((TROWS, W), jnp.float32),
)


def _sc_scores_body(emb, cen_h, ctx_h, neg_h, cen_o, ctx_o, neg_o, out,
                    cidx, xidx, nidx, coff, xoff, noff,
                    urows, vrows, nrows, pbuf, nb0, nb1, nb2, sem):
    wid = lax.axis_index("s") * NC + lax.axis_index("c")
    lanes = lax.iota(jnp.int32, 16)
    for c in range(NCHUNK):
        base = wid * PER_W + c * CHUNK
        pltpu.sync_copy(cen_h.at[pl.ds(base, CHUNK)], cidx)
        pltpu.sync_copy(ctx_h.at[pl.ds(base, CHUNK)], xidx)
        pltpu.sync_copy(neg_h.at[pl.ds(base * K, CHUNK * K)], nidx)
        pltpu.sync_copy(cen_o.at[pl.ds(base, CHUNK)], coff)
        pltpu.sync_copy(ctx_o.at[pl.ds(base, CHUNK)], xoff)
        pltpu.sync_copy(neg_o.at[pl.ds(base * K, CHUNK * K)], noff)
        d1 = pltpu.async_copy(emb.at[cidx], urows, sem)
        d2 = pltpu.async_copy(emb.at[xidx], vrows, sem)
        d3 = pltpu.async_copy(emb.at[nidx.at[pl.ds(0, 128)]],
                              nrows.at[pl.ds(0, 128)], sem)
        d4 = pltpu.async_copy(emb.at[nidx.at[pl.ds(128, 128)]],
                              nrows.at[pl.ds(128, 128)], sem)
        d5 = pltpu.async_copy(emb.at[nidx.at[pl.ds(256, 128)]],
                              nrows.at[pl.ds(256, 128)], sem)
        d1.wait(); d2.wait(); d3.wait(); d4.wait(); d5.wait()

        zero = jnp.zeros((16,), jnp.float32)

        def gbody(g, carry):
            rows = lanes + g * 16
            rows3 = rows * K
            uoff = coff[pl.ds(g * 16, 16)]
            voff = xoff[pl.ds(g * 16, 16)]
            w0 = plsc.load_gather(noff, [rows3])
            w1 = plsc.load_gather(noff, [rows3 + 1])
            w2 = plsc.load_gather(noff, [rows3 + 2])
            pa, a0, a1, a2 = zero, zero, zero, zero
            for d in range(D):
                u = plsc.load_gather(urows, [rows, uoff + d])
                v = plsc.load_gather(vrows, [rows, voff + d])
                m0 = plsc.load_gather(nrows, [rows3, w0 + d])
                m1 = plsc.load_gather(nrows, [rows3 + 1, w1 + d])
                m2 = plsc.load_gather(nrows, [rows3 + 2, w2 + d])
                pa = pa + u * v
                a0 = a0 + u * m0
                a1 = a1 + u * m1
                a2 = a2 + u * m2
            pbuf[pl.ds(g * 16, 16)] = pa
            nb0[pl.ds(g * 16, 16)] = a0
            nb1[pl.ds(g * 16, 16)] = a1
            nb2[pl.ds(g * 16, 16)] = a2
            return carry

        lax.fori_loop(0, CHUNK // 16, gbody, 0)
        pltpu.sync_copy(pbuf, out.at[0, pl.ds(base, CHUNK)])
        pltpu.sync_copy(nb0, out.at[1, pl.ds(base, CHUNK)])
        pltpu.sync_copy(nb1, out.at[2, pl.ds(base, CHUNK)])
        pltpu.sync_copy(nb2, out.at[3, pl.ds(base, CHUNK)])


_sc_scores = functools.partial(
    pl.kernel,
    out_type=jax.ShapeDtypeStruct((K + 1, B), jnp.float32),
    mesh=plsc.VectorSubcoreMesh(
        core_axis_name="c", subcore_axis_name="s",
        num_cores=NC, num_subcores=NS),
    compiler_params=pltpu.CompilerParams(needs_layout_passes=False),
    scratch_types=[
        pltpu.VMEM((CHUNK,), jnp.int32),
        pltpu.VMEM((CHUNK,), jnp.int32),
        pltpu.VMEM((CHUNK * K,), jnp.int32),
        pltpu.VMEM((CHUNK,), jnp.int32),
        pltpu.VMEM((CHUNK,), jnp.int32),
        pltpu.VMEM((CHUNK * K,), jnp.int32),
        pltpu.VMEM((CHUNK, W), jnp.float32),
        pltpu.VMEM((CHUNK, W), jnp.float32),
        pltpu.VMEM((CHUNK * K, W), jnp.float32),
        pltpu.VMEM((CHUNK,), jnp.float32),
        pltpu.VMEM((CHUNK,), jnp.float32),
        pltpu.VMEM((CHUNK,), jnp.float32),
        pltpu.VMEM((CHUNK,), jnp.float32),
        pltpu.SemaphoreType.DMA,
    ],
)(_sc_scores_body)


def _loss_body(s_ref, o_ref):
    x = s_ref[...]
    row = lax.broadcasted_iota(jnp.int32, x.shape, 0)
    ispos = row == 0
    s = jnp.where(ispos, x, -x)
    ls = jnp.minimum(s, 0.0) - jnp.log1p(jnp.exp(-jnp.abs(s)))
    pos_sum = jnp.sum(jnp.where(ispos, ls, 0.0))
    neg_sum = jnp.sum(jnp.where(ispos, 0.0, ls))
    o_ref[0, 0] = -(pos_sum / B) - (neg_sum / (K * B))


_loss = pl.pallas_call(
    _loss_body,
    out_shape=jax.ShapeDtypeStruct((1, 1), jnp.float32),
    out_specs=pl.BlockSpec(memory_space=pltpu.SMEM),
)


def _pack_row(i):
    return ((i >> 8) << 7) + (i & 127)


def _pack_off(i):
    return (i & 128) >> 1


@jax.jit
def _impl(centers, contexts, neg_samples, emb):
    cen = centers.astype(jnp.int32)
    ctx = contexts.astype(jnp.int32)
    neg = neg_samples.astype(jnp.int32).reshape(-1)
    embT = jnp.transpose(emb)
    table = _transpose(embT, embT)
    scores = _sc_scores(table, _pack_row(cen), _pack_row(ctx),
                        _pack_row(neg), _pack_off(cen), _pack_off(ctx),
                        _pack_off(neg))
    return _loss(scores)[0, 0]


def kernel(centers, contexts, neg_samples, emb):
    return _impl(centers, contexts, neg_samples, emb)


# speedup vs baseline: 4.4734x; 4.4734x over previous
import functools

import jax
import jax.numpy as jnp
from jax import lax
from jax.experimental import pallas as pl
from jax.experimental.pallas import tpu as pltpu
from jax.experimental.pallas import tpu_sc as plsc

B = 16384
D = 64
K = 3
NC = 2
NS = 16
NW = NC * NS
PER_W = B // NW
CHUNK = 128
NCHUNK = PER_W // CHUNK
W = 2 * D
SB = 1024
NBLK = 489
TROWS = NBLK * SB


def _tp_body(a_ref, b_ref, o_ref):
    o_ref[...] = jnp.concatenate(
        [a_ref[...].T, b_ref[...].T], axis=1)


_transpose = pl.pallas_call(
    _tp_body,
    grid=(NBLK,),
    in_specs=[
        pl.BlockSpec((D, SB), lambda j: (0, jnp.minimum(2 * j, 976))),
        pl.BlockSpec((D, SB), lambda j: (0, jnp.minimum(2 * j + 1, 976))),
    ],
    out_specs=pl.BlockSpec((SB, W), lambda j: (j, 0)),
    out_shape=jax.ShapeDtypeStruct((TROWS, W), jnp.float32),
)


def _sc_scores_body(emb, cen_h, ctx_h, neg_h, cen_o, ctx_o, neg_o, out,
                    cidx, xidx, nidx, coff, xoff, noff,
                    urows, vrows, nrows, pbuf, nb0, nb1, nb2, sem):
    wid = lax.axis_index("s") * NC + lax.axis_index("c")
    lanes = lax.iota(jnp.int32, 16)
    for c in range(NCHUNK):
        base = wid * PER_W + c * CHUNK
        pltpu.sync_copy(cen_h.at[pl.ds(base, CHUNK)], cidx)
        pltpu.sync_copy(ctx_h.at[pl.ds(base, CHUNK)], xidx)
        pltpu.sync_copy(neg_h.at[pl.ds(base * K, CHUNK * K)], nidx)
        pltpu.sync_copy(cen_o.at[pl.ds(base, CHUNK)], coff)
        pltpu.sync_copy(ctx_o.at[pl.ds(base, CHUNK)], xoff)
        pltpu.sync_copy(neg_o.at[pl.ds(base * K, CHUNK * K)], noff)
        d1 = pltpu.async_copy(emb.at[cidx], urows, sem)
        d2 = pltpu.async_copy(emb.at[xidx], vrows, sem)
        d3 = pltpu.async_copy(emb.at[nidx.at[pl.ds(0, 128)]],
                              nrows.at[pl.ds(0, 128)], sem)
        d4 = pltpu.async_copy(emb.at[nidx.at[pl.ds(128, 128)]],
                              nrows.at[pl.ds(128, 128)], sem)
        d5 = pltpu.async_copy(emb.at[nidx.at[pl.ds(256, 128)]],
                              nrows.at[pl.ds(256, 128)], sem)
        d1.wait(); d2.wait(); d3.wait(); d4.wait(); d5.wait()

        zero = jnp.zeros((16,), jnp.float32)

        def gbody(g, carry):
            rows = lanes + g * 16
            rows3 = rows * K
            uoff = coff[pl.ds(g * 16, 16)]
            voff = xoff[pl.ds(g * 16, 16)]
            w0 = plsc.load_gather(noff, [rows3])
            w1 = plsc.load_gather(noff, [rows3 + 1])
            w2 = plsc.load_gather(noff, [rows3 + 2])
            pa, a0, a1, a2 = zero, zero, zero, zero
            for d in range(D):
                u = plsc.load_gather(urows, [rows, uoff + d])
                v = plsc.load_gather(vrows, [rows, voff + d])
                m0 = plsc.load_gather(nrows, [rows3, w0 + d])
                m1 = plsc.load_gather(nrows, [rows3 + 1, w1 + d])
                m2 = plsc.load_gather(nrows, [rows3 + 2, w2 + d])
                pa = pa + u * v
                a0 = a0 + u * m0
                a1 = a1 + u * m1
                a2 = a2 + u * m2
            pbuf[pl.ds(g * 16, 16)] = pa
            nb0[pl.ds(g * 16, 16)] = a0
            nb1[pl.ds(g * 16, 16)] = a1
            nb2[pl.ds(g * 16, 16)] = a2
            return carry

        lax.fori_loop(0, CHUNK // 16, gbody, 0)
        pltpu.sync_copy(pbuf, out.at[0, pl.ds(base, CHUNK)])
        pltpu.sync_copy(nb0, out.at[1, pl.ds(base, CHUNK)])
        pltpu.sync_copy(nb1, out.at[2, pl.ds(base, CHUNK)])
        pltpu.sync_copy(nb2, out.at[3, pl.ds(base, CHUNK)])


_sc_scores = functools.partial(
    pl.kernel,
    out_type=jax.ShapeDtypeStruct((K + 1, B), jnp.float32),
    mesh=plsc.VectorSubcoreMesh(
        core_axis_name="c", subcore_axis_name="s",
        num_cores=NC, num_subcores=NS),
    compiler_params=pltpu.CompilerParams(needs_layout_passes=False),
    scratch_types=[
        pltpu.VMEM((CHUNK,), jnp.int32),
        pltpu.VMEM((CHUNK,), jnp.int32),
        pltpu.VMEM((CHUNK * K,), jnp.int32),
        pltpu.VMEM((CHUNK,), jnp.int32),
        pltpu.VMEM((CHUNK,), jnp.int32),
        pltpu.VMEM((CHUNK * K,), jnp.int32),
        pltpu.VMEM((CHUNK, W), jnp.float32),
        pltpu.VMEM((CHUNK, W), jnp.float32),
        pltpu.VMEM((CHUNK * K, W), jnp.float32),
        pltpu.VMEM((CHUNK,), jnp.float32),
        pltpu.VMEM((CHUNK,), jnp.float32),
        pltpu.VMEM((CHUNK,), jnp.float32),
        pltpu.VMEM((CHUNK,), jnp.float32),
        pltpu.SemaphoreType.DMA,
    ],
)(_sc_scores_body)


def _loss_body(s_ref, o_ref):
    x = s_ref[...]
    row = lax.broadcasted_iota(jnp.int32, x.shape, 0)
    ispos = row == 0
    s = jnp.where(ispos, x, -x)
    ls = jnp.minimum(s, 0.0) - jnp.log1p(jnp.exp(-jnp.abs(s)))
    pos_sum = jnp.sum(jnp.where(ispos, ls, 0.0))
    neg_sum = jnp.sum(jnp.where(ispos, 0.0, ls))
    o_ref[0, 0] = -(pos_sum / B) - (neg_sum / (K * B))


_loss = pl.pallas_call(
    _loss_body,
    out_shape=jax.ShapeDtypeStruct((1, 1), jnp.float32),
    out_specs=pl.BlockSpec(memory_space=pltpu.SMEM),
)


def _pack_row(i):
    return ((i >> 11) << 10) + (i & (SB - 1))


def _pack_off(i):
    return ((i >> 10) & 1) << 6


@jax.jit
def _impl(centers, contexts, neg_samples, emb):
    cen = centers.astype(jnp.int32)
    ctx = contexts.astype(jnp.int32)
    neg = neg_samples.astype(jnp.int32).reshape(-1)
    embT = jnp.transpose(emb)
    table = _transpose(embT, embT)
    scores = _sc_scores(table, _pack_row(cen), _pack_row(ctx),
                        _pack_row(neg), _pack_off(cen), _pack_off(ctx),
                        _pack_off(neg))
    return _loss(scores)[0, 0]


def kernel(centers, contexts, neg_samples, emb):
    return _impl(centers, contexts, neg_samples, emb)


# speedup vs baseline: 5.5944x; 1.2506x over previous
import functools

import jax
import jax.numpy as jnp
from jax import lax
from jax.experimental import pallas as pl
from jax.experimental.pallas import tpu as pltpu
from jax.experimental.pallas import tpu_sc as plsc

B = 16384
D = 64
K = 3
NC = 2
NS = 16
NW = NC * NS
PER_W = B // NW
CHUNK = 128
NCHUNK = PER_W // CHUNK
W = 2 * D
SB = 2048
NBLK = 245
TROWS = NBLK * SB


def _tp_body(a_ref, b_ref, o_ref):
    o_ref[...] = jnp.concatenate(
        [a_ref[...].T, b_ref[...].T], axis=1)


_transpose = pl.pallas_call(
    _tp_body,
    grid=(NBLK,),
    in_specs=[
        pl.BlockSpec((D, SB), lambda j: (0, jnp.minimum(2 * j, 488))),
        pl.BlockSpec((D, SB), lambda j: (0, jnp.minimum(2 * j + 1, 488))),
    ],
    out_specs=pl.BlockSpec((SB, W), lambda j: (j, 0)),
    out_shape=jax.ShapeDtypeStruct((TROWS, W), jnp.float32),
)


def _sc_scores_body(emb, cen_h, ctx_h, neg_h, cen_o, ctx_o, neg_o, out,
                    cidx, xidx, nidx, coff, xoff, noff,
                    urows, vrows, nrows, pbuf, nb0, nb1, nb2, sem):
    wid = lax.axis_index("s") * NC + lax.axis_index("c")
    lanes = lax.iota(jnp.int32, 16)
    for c in range(NCHUNK):
        base = wid * PER_W + c * CHUNK
        pltpu.sync_copy(cen_h.at[pl.ds(base, CHUNK)], cidx)
        pltpu.sync_copy(ctx_h.at[pl.ds(base, CHUNK)], xidx)
        pltpu.sync_copy(neg_h.at[pl.ds(base * K, CHUNK * K)], nidx)
        pltpu.sync_copy(cen_o.at[pl.ds(base, CHUNK)], coff)
        pltpu.sync_copy(ctx_o.at[pl.ds(base, CHUNK)], xoff)
        pltpu.sync_copy(neg_o.at[pl.ds(base * K, CHUNK * K)], noff)
        d1 = pltpu.async_copy(emb.at[cidx], urows, sem)
        d2 = pltpu.async_copy(emb.at[xidx], vrows, sem)
        d3 = pltpu.async_copy(emb.at[nidx.at[pl.ds(0, 128)]],
                              nrows.at[pl.ds(0, 128)], sem)
        d4 = pltpu.async_copy(emb.at[nidx.at[pl.ds(128, 128)]],
                              nrows.at[pl.ds(128, 128)], sem)
        d5 = pltpu.async_copy(emb.at[nidx.at[pl.ds(256, 128)]],
                              nrows.at[pl.ds(256, 128)], sem)
        d1.wait(); d2.wait(); d3.wait(); d4.wait(); d5.wait()

        zero = jnp.zeros((16,), jnp.float32)

        def gbody(g, carry):
            rows = lanes + g * 16
            rows3 = rows * K
            uoff = coff[pl.ds(g * 16, 16)]
            voff = xoff[pl.ds(g * 16, 16)]
            w0 = plsc.load_gather(noff, [rows3])
            w1 = plsc.load_gather(noff, [rows3 + 1])
            w2 = plsc.load_gather(noff, [rows3 + 2])
            pa, a0, a1, a2 = zero, zero, zero, zero
            for d in range(D):
                u = plsc.load_gather(urows, [rows, uoff + d])
                v = plsc.load_gather(vrows, [rows, voff + d])
                m0 = plsc.load_gather(nrows, [rows3, w0 + d])
                m1 = plsc.load_gather(nrows, [rows3 + 1, w1 + d])
                m2 = plsc.load_gather(nrows, [rows3 + 2, w2 + d])
                pa = pa + u * v
                a0 = a0 + u * m0
                a1 = a1 + u * m1
                a2 = a2 + u * m2
            pbuf[pl.ds(g * 16, 16)] = pa
            nb0[pl.ds(g * 16, 16)] = a0
            nb1[pl.ds(g * 16, 16)] = a1
            nb2[pl.ds(g * 16, 16)] = a2
            return carry

        lax.fori_loop(0, CHUNK // 16, gbody, 0)
        pltpu.sync_copy(pbuf, out.at[0, pl.ds(base, CHUNK)])
        pltpu.sync_copy(nb0, out.at[1, pl.ds(base, CHUNK)])
        pltpu.sync_copy(nb1, out.at[2, pl.ds(base, CHUNK)])
        pltpu.sync_copy(nb2, out.at[3, pl.ds(base, CHUNK)])


_sc_scores = functools.partial(
    pl.kernel,
    out_type=jax.ShapeDtypeStruct((K + 1, B), jnp.float32),
    mesh=plsc.VectorSubcoreMesh(
        core_axis_name="c", subcore_axis_name="s",
        num_cores=NC, num_subcores=NS),
    compiler_params=pltpu.CompilerParams(needs_layout_passes=False),
    scratch_types=[
        pltpu.VMEM((CHUNK,), jnp.int32),
        pltpu.VMEM((CHUNK,), jnp.int32),
        pltpu.VMEM((CHUNK * K,), jnp.int32),
        pltpu.VMEM((CHUNK,), jnp.int32),
        pltpu.VMEM((CHUNK,), jnp.int32),
        pltpu.VMEM((CHUNK * K,), jnp.int32),
        pltpu.VMEM((CHUNK, W), jnp.float32),
        pltpu.VMEM((CHUNK, W), jnp.float32),
        pltpu.VMEM((CHUNK * K, W), jnp.float32),
        pltpu.VMEM((CHUNK,), jnp.float32),
        pltpu.VMEM((CHUNK,), jnp.float32),
        pltpu.VMEM((CHUNK,), jnp.float32),
        pltpu.VMEM((CHUNK,), jnp.float32),
        pltpu.SemaphoreType.DMA,
    ],
)(_sc_scores_body)


def _loss_body(s_ref, o_ref):
    x = s_ref[...]
    row = lax.broadcasted_iota(jnp.int32, x.shape, 0)
    ispos = row == 0
    s = jnp.where(ispos, x, -x)
    ls = jnp.minimum(s, 0.0) - jnp.log1p(jnp.exp(-jnp.abs(s)))
    pos_sum = jnp.sum(jnp.where(ispos, ls, 0.0))
    neg_sum = jnp.sum(jnp.where(ispos, 0.0, ls))
    o_ref[0, 0] = -(pos_sum / B) - (neg_sum / (K * B))


_loss = pl.pallas_call(
    _loss_body,
    out_shape=jax.ShapeDtypeStruct((1, 1), jnp.float32),
    out_specs=pl.BlockSpec(memory_space=pltpu.SMEM),
)


def _pack_row(i):
    return ((i >> 12) << 11) + (i & (SB - 1))


def _pack_off(i):
    return ((i >> 11) & 1) << 6


@jax.jit
def _impl(centers, contexts, neg_samples, emb):
    cen = centers.astype(jnp.int32)
    ctx = contexts.astype(jnp.int32)
    neg = neg_samples.astype(jnp.int32).reshape(-1)
    embT = jnp.transpose(emb)
    table = _transpose(embT, embT)
    scores = _sc_scores(table, _pack_row(cen), _pack_row(ctx),
                        _pack_row(neg), _pack_off(cen), _pack_off(ctx),
                        _pack_off(neg))
    return _loss(scores)[0, 0]


def kernel(centers, contexts, neg_samples, emb):
    return _impl(centers, contexts, neg_samples, emb)


# speedup vs baseline: 5.7365x; 1.0254x over previous
import functools

import jax
import jax.numpy as jnp
from jax import lax
from jax.experimental import pallas as pl
from jax.experimental.pallas import tpu as pltpu
from jax.experimental.pallas import tpu_sc as plsc

B = 16384
D = 64
K = 3
NC = 2
NS = 16
NW = NC * NS
PER_W = B // NW
CHUNK = 64
NCHUNK = PER_W // CHUNK
W = 2 * D
SB = 2048
NBLK = 245
TROWS = NBLK * SB


def _tp_body(a_ref, b_ref, o_ref):
    o_ref[...] = jnp.concatenate(
        [a_ref[...].T, b_ref[...].T], axis=1)


_transpose = pl.pallas_call(
    _tp_body,
    grid=(NBLK,),
    in_specs=[
        pl.BlockSpec((D, SB), lambda j: (0, jnp.minimum(2 * j, 488))),
        pl.BlockSpec((D, SB), lambda j: (0, jnp.minimum(2 * j + 1, 488))),
    ],
    out_specs=pl.BlockSpec((SB, W), lambda j: (j, 0)),
    out_shape=jax.ShapeDtypeStruct((TROWS, W), jnp.float32),
)


BLOB = 10 * CHUNK


def _sc_scores_body(emb, blob, out,
                    ibufs, urowss, vrowss, nrowss, sbufs, sems):
    wid = lax.axis_index("s") * NC + lax.axis_index("c")
    lanes = lax.iota(jnp.int32, 16)
    zero = jnp.zeros((16,), jnp.float32)
    idx_d, gat_d, st_d = {}, {}, {}

    def fire_idx(c):
        p = c % 2
        idx_d[c] = pltpu.async_copy(blob.at[wid * NCHUNK + c],
                                    ibufs[p], sems[p])

    def fire_gathers(c):
        p = c % 2
        idx_d[c].wait()
        ibuf = ibufs[p]
        gat_d[c] = [
            pltpu.async_copy(emb.at[ibuf.at[pl.ds(0, CHUNK)]],
                             urowss[p], sems[2 + p]),
            pltpu.async_copy(emb.at[ibuf.at[pl.ds(CHUNK, CHUNK)]],
                             vrowss[p], sems[2 + p]),
            pltpu.async_copy(emb.at[ibuf.at[pl.ds(2 * CHUNK, 128)]],
                             nrowss[p].at[pl.ds(0, 128)], sems[2 + p]),
            pltpu.async_copy(emb.at[ibuf.at[pl.ds(2 * CHUNK + 128, 64)]],
                             nrowss[p].at[pl.ds(128, 64)], sems[2 + p]),
        ]

    def compute(c):
        p = c % 2
        ibuf, urows, vrows, nrows, sbuf = (
            ibufs[p], urowss[p], vrowss[p], nrowss[p], sbufs[p])
        if c - 2 >= 0:
            for d in st_d[c - 2]:
                d.wait()
        for d in gat_d[c]:
            d.wait()

        def gbody(g, carry):
            rows = lanes + g * 16
            rows3 = rows * K
            uoff = ibuf[pl.ds(5 * CHUNK + g * 16, 16)]
            voff = ibuf[pl.ds(6 * CHUNK + g * 16, 16)]
            w0 = plsc.load_gather(ibuf, [rows3 + 7 * CHUNK])
            w1 = plsc.load_gather(ibuf, [rows3 + (7 * CHUNK + 1)])
            w2 = plsc.load_gather(ibuf, [rows3 + (7 * CHUNK + 2)])
            pa, a0, a1, a2 = zero, zero, zero, zero
            for d in range(D):
                u = plsc.load_gather(urows, [rows, uoff + d])
                v = plsc.load_gather(vrows, [rows, voff + d])
                m0 = plsc.load_gather(nrows, [rows3, w0 + d])
                m1 = plsc.load_gather(nrows, [rows3 + 1, w1 + d])
                m2 = plsc.load_gather(nrows, [rows3 + 2, w2 + d])
                pa = pa + u * v
                a0 = a0 + u * m0
                a1 = a1 + u * m1
                a2 = a2 + u * m2
            sbuf[0, pl.ds(g * 16, 16)] = pa
            sbuf[1, pl.ds(g * 16, 16)] = a0
            sbuf[2, pl.ds(g * 16, 16)] = a1
            sbuf[3, pl.ds(g * 16, 16)] = a2
            return carry

        lax.fori_loop(0, CHUNK // 16, gbody, 0)
        base = wid * PER_W + c * CHUNK
        st_d[c] = [
            pltpu.async_copy(sbuf.at[k], out.at[k, pl.ds(base, CHUNK)],
                             sems[4 + p])
            for k in range(K + 1)]

    fire_idx(0)
    fire_idx(1)
    fire_gathers(0)
    for c in range(NCHUNK):
        if c + 1 < NCHUNK:
            fire_gathers(c + 1)
        compute(c)
        if c + 2 < NCHUNK:
            fire_idx(c + 2)
    for c in (NCHUNK - 2, NCHUNK - 1):
        for d in st_d[c]:
            d.wait()


_sc_scores = functools.partial(
    pl.kernel,
    out_type=jax.ShapeDtypeStruct((K + 1, B), jnp.float32),
    mesh=plsc.VectorSubcoreMesh(
        core_axis_name="c", subcore_axis_name="s",
        num_cores=NC, num_subcores=NS),
    compiler_params=pltpu.CompilerParams(needs_layout_passes=False),
    scratch_types=[
        [pltpu.VMEM((BLOB,), jnp.int32)] * 2,
        [pltpu.VMEM((CHUNK, W), jnp.float32)] * 2,
        [pltpu.VMEM((CHUNK, W), jnp.float32)] * 2,
        [pltpu.VMEM((CHUNK * K, W), jnp.float32)] * 2,
        [pltpu.VMEM((K + 1, CHUNK), jnp.float32)] * 2,
        [pltpu.SemaphoreType.DMA] * 6,
    ],
)(_sc_scores_body)


def _loss_body(s_ref, o_ref):
    x = s_ref[...]
    row = lax.broadcasted_iota(jnp.int32, x.shape, 0)
    ispos = row == 0
    s = jnp.where(ispos, x, -x)
    ls = jnp.minimum(s, 0.0) - jnp.log1p(jnp.exp(-jnp.abs(s)))
    pos_sum = jnp.sum(jnp.where(ispos, ls, 0.0))
    neg_sum = jnp.sum(jnp.where(ispos, 0.0, ls))
    o_ref[0, 0] = -(pos_sum / B) - (neg_sum / (K * B))


_loss = pl.pallas_call(
    _loss_body,
    out_shape=jax.ShapeDtypeStruct((1, 1), jnp.float32),
    out_specs=pl.BlockSpec(memory_space=pltpu.SMEM),
)


def _pack_row(i):
    return ((i >> 12) << 11) + (i & (SB - 1))


def _pack_off(i):
    return ((i >> 11) & 1) << 6


@jax.jit
def _impl(centers, contexts, neg_samples, emb):
    cen = centers.astype(jnp.int32)
    ctx = contexts.astype(jnp.int32)
    neg = neg_samples.astype(jnp.int32).reshape(-1)
    embT = jnp.transpose(emb)
    table = _transpose(embT, embT)
    blob = jnp.concatenate(
        [_pack_row(cen).reshape(-1, CHUNK),
         _pack_row(ctx).reshape(-1, CHUNK),
         _pack_row(neg).reshape(-1, K * CHUNK),
         _pack_off(cen).reshape(-1, CHUNK),
         _pack_off(ctx).reshape(-1, CHUNK),
         _pack_off(neg).reshape(-1, K * CHUNK)], axis=1)
    scores = _sc_scores(table, blob)
    return _loss(scores)[0, 0]


def kernel(centers, contexts, neg_samples, emb):
    return _impl(centers, contexts, neg_samples, emb)


# speedup vs baseline: 6.7088x; 1.1695x over previous
import functools

import jax
import jax.numpy as jnp
from jax import lax
from jax.experimental import pallas as pl
from jax.experimental.pallas import tpu as pltpu
from jax.experimental.pallas import tpu_sc as plsc

B = 16384
D = 64
K = 3
NC = 2
NS = 16
NW = NC * NS
PER_W = B // NW
CHUNK = 64
NCHUNK = PER_W // CHUNK
W = 2 * D
SB = 4096
NBLK = 123
TROWS = NBLK * SB


def _tp_body(a_ref, b_ref, o_ref):
    o_ref[...] = jnp.concatenate(
        [a_ref[...].T, b_ref[...].T], axis=1)


_transpose = pl.pallas_call(
    _tp_body,
    grid=(NBLK,),
    in_specs=[
        pl.BlockSpec((D, SB), lambda j: (0, jnp.minimum(2 * j, 244))),
        pl.BlockSpec((D, SB), lambda j: (0, jnp.minimum(2 * j + 1, 244))),
    ],
    out_specs=pl.BlockSpec((SB, W), lambda j: (j, 0)),
    out_shape=jax.ShapeDtypeStruct((TROWS, W), jnp.float32),
)


BLOB = 10 * CHUNK


def _sc_scores_body(emb, blob, out,
                    ibufs, urowss, vrowss, nrowss, sbufs, sems):
    wid = lax.axis_index("s") * NC + lax.axis_index("c")
    lanes = lax.iota(jnp.int32, 16)
    zero = jnp.zeros((16,), jnp.float32)
    idx_d, gat_d, st_d = {}, {}, {}

    def fire_idx(c):
        p = c % 2
        idx_d[c] = pltpu.async_copy(blob.at[wid * NCHUNK + c],
                                    ibufs[p], sems[p])

    def fire_gathers(c):
        p = c % 2
        idx_d[c].wait()
        ibuf = ibufs[p]
        gat_d[c] = [
            pltpu.async_copy(emb.at[ibuf.at[pl.ds(0, CHUNK)]],
                             urowss[p], sems[2 + p]),
            pltpu.async_copy(emb.at[ibuf.at[pl.ds(CHUNK, CHUNK)]],
                             vrowss[p], sems[2 + p]),
            pltpu.async_copy(emb.at[ibuf.at[pl.ds(2 * CHUNK, 128)]],
                             nrowss[p].at[pl.ds(0, 128)], sems[2 + p]),
            pltpu.async_copy(emb.at[ibuf.at[pl.ds(2 * CHUNK + 128, 64)]],
                             nrowss[p].at[pl.ds(128, 64)], sems[2 + p]),
        ]

    def compute(c):
        p = c % 2
        ibuf, urows, vrows, nrows, sbuf = (
            ibufs[p], urowss[p], vrowss[p], nrowss[p], sbufs[p])
        if c - 2 >= 0:
            for d in st_d[c - 2]:
                d.wait()
        for d in gat_d[c]:
            d.wait()

        def gbody(g, carry):
            rows = lanes + g * 16
            rows3 = rows * K
            uoff = ibuf[pl.ds(5 * CHUNK + g * 16, 16)]
            voff = ibuf[pl.ds(6 * CHUNK + g * 16, 16)]
            w0 = plsc.load_gather(ibuf, [rows3 + 7 * CHUNK])
            w1 = plsc.load_gather(ibuf, [rows3 + (7 * CHUNK + 1)])
            w2 = plsc.load_gather(ibuf, [rows3 + (7 * CHUNK + 2)])
            pa, a0, a1, a2 = zero, zero, zero, zero
            for d in range(D):
                u = plsc.load_gather(urows, [rows, uoff + d])
                v = plsc.load_gather(vrows, [rows, voff + d])
                m0 = plsc.load_gather(nrows, [rows3, w0 + d])
                m1 = plsc.load_gather(nrows, [rows3 + 1, w1 + d])
                m2 = plsc.load_gather(nrows, [rows3 + 2, w2 + d])
                pa = pa + u * v
                a0 = a0 + u * m0
                a1 = a1 + u * m1
                a2 = a2 + u * m2
            sbuf[0, pl.ds(g * 16, 16)] = pa
            sbuf[1, pl.ds(g * 16, 16)] = a0
            sbuf[2, pl.ds(g * 16, 16)] = a1
            sbuf[3, pl.ds(g * 16, 16)] = a2
            return carry

        lax.fori_loop(0, CHUNK // 16, gbody, 0)
        base = wid * PER_W + c * CHUNK
        st_d[c] = [
            pltpu.async_copy(sbuf.at[k], out.at[k, pl.ds(base, CHUNK)],
                             sems[4 + p])
            for k in range(K + 1)]

    fire_idx(0)
    fire_idx(1)
    fire_gathers(0)
    for c in range(NCHUNK):
        if c + 1 < NCHUNK:
            fire_gathers(c + 1)
        compute(c)
        if c + 2 < NCHUNK:
            fire_idx(c + 2)
    for c in (NCHUNK - 2, NCHUNK - 1):
        for d in st_d[c]:
            d.wait()


_sc_scores = functools.partial(
    pl.kernel,
    out_type=jax.ShapeDtypeStruct((K + 1, B), jnp.float32),
    mesh=plsc.VectorSubcoreMesh(
        core_axis_name="c", subcore_axis_name="s",
        num_cores=NC, num_subcores=NS),
    compiler_params=pltpu.CompilerParams(needs_layout_passes=False),
    scratch_types=[
        [pltpu.VMEM((BLOB,), jnp.int32)] * 2,
        [pltpu.VMEM((CHUNK, W), jnp.float32)] * 2,
        [pltpu.VMEM((CHUNK, W), jnp.float32)] * 2,
        [pltpu.VMEM((CHUNK * K, W), jnp.float32)] * 2,
        [pltpu.VMEM((K + 1, CHUNK), jnp.float32)] * 2,
        [pltpu.SemaphoreType.DMA] * 6,
    ],
)(_sc_scores_body)


def _loss_body(s_ref, o_ref):
    x = s_ref[...]
    row = lax.broadcasted_iota(jnp.int32, x.shape, 0)
    ispos = row == 0
    s = jnp.where(ispos, x, -x)
    ls = jnp.minimum(s, 0.0) - jnp.log1p(jnp.exp(-jnp.abs(s)))
    pos_sum = jnp.sum(jnp.where(ispos, ls, 0.0))
    neg_sum = jnp.sum(jnp.where(ispos, 0.0, ls))
    o_ref[0, 0] = -(pos_sum / B) - (neg_sum / (K * B))


_loss = pl.pallas_call(
    _loss_body,
    out_shape=jax.ShapeDtypeStruct((1, 1), jnp.float32),
    out_specs=pl.BlockSpec(memory_space=pltpu.SMEM),
)


def _pack_row(i):
    return ((i >> 13) << 12) + (i & (SB - 1))


def _pack_off(i):
    return ((i >> 12) & 1) << 6


@jax.jit
def _impl(centers, contexts, neg_samples, emb):
    cen = centers.astype(jnp.int32)
    ctx = contexts.astype(jnp.int32)
    neg = neg_samples.astype(jnp.int32).reshape(-1)
    embT = jnp.transpose(emb)
    table = _transpose(embT, embT)
    blob = jnp.concatenate(
        [_pack_row(cen).reshape(-1, CHUNK),
         _pack_row(ctx).reshape(-1, CHUNK),
         _pack_row(neg).reshape(-1, K * CHUNK),
         _pack_off(cen).reshape(-1, CHUNK),
         _pack_off(ctx).reshape(-1, CHUNK),
         _pack_off(neg).reshape(-1, K * CHUNK)], axis=1)
    scores = _sc_scores(table, blob)
    return _loss(scores)[0, 0]


def kernel(centers, contexts, neg_samples, emb):
    return _impl(centers, contexts, neg_samples, emb)
